# baseline (speedup 1.0000x reference)
import math

import jax
import jax.numpy as jnp
from jax.experimental import pallas as pl
from jax.experimental.pallas import tpu as pltpu

B, N, T = 16, 512, 3
HID, H, C = 128, 4, 32
E = 65536

ECHUNK = 2048
NCHUNK = E // ECHUNK


def _gelu(v):
    return 0.5 * v * (1.0 + jax.lax.erf(v * (1.0 / math.sqrt(2.0))))


def _cnt_kernel(src_ref, dst_ref, out_ref):
    @pl.when(pl.program_id(0) == 0)
    def _():
        out_ref[...] = jnp.zeros_like(out_ref)

    s_blk = src_ref[0]
    d_blk = dst_ref[0]
    iota = jax.lax.broadcasted_iota(jnp.int32, (N, 1), 0)
    oh_s = (iota == s_blk).astype(jnp.bfloat16)
    oh_d = (iota == d_blk).astype(jnp.bfloat16)
    part = jax.lax.dot_general(
        oh_d, oh_s, (((1,), (1,)), ((), ())),
        preferred_element_type=jnp.float32)
    out_ref[...] += part


def _edge_counts(edge_index):
    src = edge_index[0].reshape(NCHUNK, 1, ECHUNK)
    dst = edge_index[1].reshape(NCHUNK, 1, ECHUNK)
    return pl.pallas_call(
        _cnt_kernel,
        grid=(NCHUNK,),
        in_specs=[
            pl.BlockSpec((1, 1, ECHUNK), lambda i: (i, 0, 0)),
            pl.BlockSpec((1, 1, ECHUNK), lambda i: (i, 0, 0)),
        ],
        out_specs=pl.BlockSpec((N, N), lambda i: (0, 0)),
        out_shape=jax.ShapeDtypeStruct((N, N), jnp.float32),
    )(src, dst)


def _gat_layer(h, cntT, neg_maskT, Wl, bl, Wr, br, kpos_ref, Pun, bias,
               xlpT_scr, xrpT_scr, att_flat):
    xl = jnp.dot(h, Wl, preferred_element_type=jnp.float32) + bl
    xr = jnp.dot(h, Wr, preferred_element_type=jnp.float32) + br
    xlpT_scr[...] = (xl * att_flat).T
    xrpT_scr[...] = (xr * att_flat).T
    ones_row = jnp.ones((1, N), jnp.float32)
    ones_c = jnp.ones((1, C), jnp.float32)
    absmask = jnp.int32(0x7FFFFFFF)
    outs = []
    for hd in range(H):
        xl_h = xl[:, hd * C:(hd + 1) * C]
        al_row = jnp.dot(ones_c, xlpT_scr[hd * C:(hd + 1) * C, :],
                         preferred_element_type=jnp.float32)
        ar_row = jnp.dot(ones_c, xrpT_scr[hd * C:(hd + 1) * C, :],
                         preferred_element_type=jnp.float32)
        lin = jax.lax.dot_general(
            jnp.concatenate([1.5 * ar_row, ones_row], axis=0),
            jnp.concatenate([ones_row, 1.5 * al_row], axis=0),
            (((0,), (0,)), ((), ())),
            preferred_element_type=jnp.float32)

        def absval(c):
            vp = xrpT_scr[pl.ds(hd * C + c, 1), :]
            up = xlpT_scr[pl.ds(hd * C + c, 1), :]
            tp = jax.lax.dot_general(
                jnp.concatenate([vp, ones_row], axis=0),
                jnp.concatenate([ones_row, up], axis=0),
                (((0,), (0,)), ((), ())),
                preferred_element_type=jnp.float32)
            ti = jax.lax.bitcast_convert_type(tp, jnp.int32) & absmask
            return jax.lax.bitcast_convert_type(ti, jnp.float32)

        kp = kpos_ref[hd]
        m = jax.lax.fori_loop(0, kp, lambda c, a: a + absval(c), lin)
        m = jax.lax.fori_loop(kp, C, lambda c, a: a - absval(c), m)
        alphaT = 0.4 * m
        amax = jnp.max(alphaT + neg_maskT, axis=1, keepdims=True)
        amax = jnp.where(jnp.isfinite(amax), amax, 0.0)
        ex = jnp.exp(jnp.minimum(alphaT - amax, 0.0))
        w = cntT * ex
        den = jnp.sum(w, axis=1, keepdims=True)
        wn = w * (1.0 / (den + 1e-16))
        out_h = jnp.dot(wn, xl_h, preferred_element_type=jnp.float32)
        outs.append(out_h)
    cat = jnp.concatenate(outs, axis=1)
    return jnp.dot(cat, Pun, preferred_element_type=jnp.float32) + bias


def _main_kernel(kpos1_ref, kpos2_ref,
                 xb_ref, cntT_ref,
                 W0t_ref, b0_ref, g0_ref, be0_ref,
                 Wl1_ref, bl1_ref, Wr1_ref, br1_ref, af1_ref, Pun1_ref,
                 bias1_ref,
                 Wl2_ref, bl2_ref, Wr2_ref, br2_ref, af2_ref, Pun2_ref,
                 bias2_ref,
                 Wout_ref, bout_ref, g1_ref, be1_ref,
                 out_ref, xlpT_scr, xrpT_scr):
    eps = 1e-5
    inv = 1.0 / math.sqrt(1.0 + eps)
    xb = xb_ref[0]
    cntT = cntT_ref[...]
    neg_maskT = jnp.where(cntT > 0.0, 0.0, -jnp.inf)

    z = jnp.dot(xb, W0t_ref[...], preferred_element_type=jnp.float32)
    z = z + b0_ref[...]
    z = g0_ref[...] * (z * inv) + be0_ref[...]
    h = _gelu(z)

    h = _gat_layer(h, cntT, neg_maskT, Wl1_ref[...], bl1_ref[...],
                   Wr1_ref[...], br1_ref[...], kpos1_ref, Pun1_ref[...],
                   bias1_ref[...], xlpT_scr, xrpT_scr, af1_ref[...])
    h = _gelu(h)
    h = _gat_layer(h, cntT, neg_maskT, Wl2_ref[...], bl2_ref[...],
                   Wr2_ref[...], br2_ref[...], kpos2_ref, Pun2_ref[...],
                   bias2_ref[...], xlpT_scr, xrpT_scr, af2_ref[...])
    h = _gelu(h)

    pooled = jnp.mean(h, axis=0, keepdims=True)
    o = jnp.dot(pooled, Wout_ref[...], preferred_element_type=jnp.float32)
    o = o + bout_ref[...]
    o = g1_ref[...] * (o * inv) + be1_ref[...]
    out_ref[0] = _gelu(o)


def _permute_head_channels(Wl, bl, Wr, br, att):
    neg = att < 0.0
    order = jnp.argsort(neg.astype(jnp.int32), axis=1, stable=True)
    flatperm = (order + jnp.arange(H, dtype=jnp.int32)[:, None] * C).reshape(HID)
    kpos = jnp.sum(~neg, axis=1).astype(jnp.int32)
    Pun = (flatperm[:, None] ==
           jnp.arange(HID, dtype=jnp.int32)[None, :]).astype(jnp.float32)
    return (Wl[:, flatperm], bl[flatperm], Wr[:, flatperm], br[flatperm],
            att.reshape(HID)[flatperm].reshape(1, HID), kpos, Pun)


def kernel(x, edge_index, W0, b0, g0, be0, Wl1, bl1, Wr1, br1, att1, bias1,
           Wl2, bl2, Wr2, br2, att2, bias2, Wout, bout, g1, be1):
    cntT = _edge_counts(edge_index)

    xb = x.reshape(B, N, N * T)
    row = lambda v: v.reshape(1, HID)
    full = lambda a: pl.BlockSpec(a.shape, lambda b: (0,) * a.ndim)

    Wl1p, bl1p, Wr1p, br1p, af1, kpos1, Pun1 = _permute_head_channels(
        Wl1, bl1, Wr1, br1, att1)
    Wl2p, bl2p, Wr2p, br2p, af2, kpos2, Pun2 = _permute_head_channels(
        Wl2, bl2, Wr2, br2, att2)

    smem_spec = pl.BlockSpec(memory_space=pltpu.SMEM)
    args = [kpos1, kpos2,
            xb, cntT,
            W0.T, row(b0), row(g0), row(be0),
            Wl1p, row(bl1p), Wr1p, row(br1p), af1, Pun1, row(bias1),
            Wl2p, row(bl2p), Wr2p, row(br2p), af2, Pun2, row(bias2),
            Wout, row(bout), row(g1), row(be1)]
    in_specs = [smem_spec, smem_spec,
                pl.BlockSpec((1, N, N * T), lambda b: (b, 0, 0))]
    in_specs += [full(a) for a in args[3:]]

    out = pl.pallas_call(
        _main_kernel,
        grid=(B,),
        in_specs=in_specs,
        out_specs=pl.BlockSpec((1, 1, HID), lambda b: (b, 0, 0)),
        out_shape=jax.ShapeDtypeStruct((B, 1, HID), jnp.float32),
        scratch_shapes=[
            pltpu.VMEM((HID, N), jnp.float32),
            pltpu.VMEM((HID, N), jnp.float32),
        ],
        compiler_params=pltpu.CompilerParams(
            dimension_semantics=("parallel",)),
    )(*args)
    return out.reshape(B, HID)


# speedup vs baseline: 1.9211x; 1.9211x over previous
import math

import jax
import jax.numpy as jnp
from jax.experimental import pallas as pl
from jax.experimental.pallas import tpu as pltpu

B, N, T = 16, 512, 3
HID, H, C = 128, 4, 32
E = 65536

ECHUNK = 2048
NCHUNK = E // ECHUNK


def _gelu(v):
    return 0.5 * v * (1.0 + jax.lax.erf(v * (1.0 / math.sqrt(2.0))))


def _cnt_kernel(src_ref, dst_ref, out_ref):
    @pl.when(pl.program_id(0) == 0)
    def _():
        out_ref[...] = jnp.zeros_like(out_ref)

    s_blk = src_ref[0]
    d_blk = dst_ref[0]
    iota = jax.lax.broadcasted_iota(jnp.int32, (N, 1), 0)
    oh_s = (iota == s_blk).astype(jnp.bfloat16)
    oh_d = (iota == d_blk).astype(jnp.bfloat16)
    part = jax.lax.dot_general(
        oh_d, oh_s, (((1,), (1,)), ((), ())),
        preferred_element_type=jnp.float32)
    out_ref[...] += part


def _edge_counts(edge_index):
    src = edge_index[0].reshape(NCHUNK, 1, ECHUNK)
    dst = edge_index[1].reshape(NCHUNK, 1, ECHUNK)
    return pl.pallas_call(
        _cnt_kernel,
        grid=(NCHUNK,),
        in_specs=[
            pl.BlockSpec((1, 1, ECHUNK), lambda i: (i, 0, 0)),
            pl.BlockSpec((1, 1, ECHUNK), lambda i: (i, 0, 0)),
        ],
        out_specs=pl.BlockSpec((N, N), lambda i: (0, 0)),
        out_shape=jax.ShapeDtypeStruct((N, N), jnp.float32),
    )(src, dst)


def _gat_layer(h, cntT, neg_maskT, Wl, bl, Wr, br, sgn_ref, bias,
               xlpT_scr, xrpT_scr, att_flat):
    xl = jnp.dot(h, Wl, preferred_element_type=jnp.float32) + bl
    xr = jnp.dot(h, Wr, preferred_element_type=jnp.float32) + br
    xlpT_scr[...] = (xl * att_flat).T
    xrpT_scr[...] = (xr * att_flat).T
    ones_row = jnp.ones((1, N), jnp.float32)
    ones_c = jnp.ones((1, C), jnp.float32)
    absmask = jnp.int32(0x7FFFFFFF)
    outs = []
    for hd in range(H):
        xl_h = xl[:, hd * C:(hd + 1) * C]
        al_row = jnp.dot(ones_c, xlpT_scr[hd * C:(hd + 1) * C, :],
                         preferred_element_type=jnp.float32)
        ar_row = jnp.dot(ones_c, xrpT_scr[hd * C:(hd + 1) * C, :],
                         preferred_element_type=jnp.float32)
        lin = jax.lax.dot_general(
            jnp.concatenate([1.5 * ar_row, ones_row], axis=0),
            jnp.concatenate([ones_row, 1.5 * al_row], axis=0),
            (((0,), (0,)), ((), ())),
            preferred_element_type=jnp.float32)

        def body(c, acc):
            vp = xrpT_scr[pl.ds(hd * C + c, 1), :]
            up = xlpT_scr[pl.ds(hd * C + c, 1), :]
            tp = jax.lax.dot_general(
                jnp.concatenate([vp, ones_row], axis=0),
                jnp.concatenate([ones_row, up], axis=0),
                (((0,), (0,)), ((), ())),
                preferred_element_type=jnp.float32)
            ti = (jax.lax.bitcast_convert_type(tp, jnp.int32) & absmask) \
                ^ sgn_ref[pl.ds(hd * C + c, 1), 0:1]
            return acc + jax.lax.bitcast_convert_type(ti, jnp.float32)

        m = jax.lax.fori_loop(0, C, body, lin, unroll=4)
        alphaT = 0.4 * m
        amax = jnp.max(alphaT + neg_maskT, axis=1, keepdims=True)
        amax = jnp.where(jnp.isfinite(amax), amax, 0.0)
        ex = jnp.exp(jnp.minimum(alphaT - amax, 0.0))
        w = cntT * ex
        den = jnp.sum(w, axis=1, keepdims=True)
        wn = w * (1.0 / (den + 1e-16))
        out_h = jnp.dot(wn, xl_h, preferred_element_type=jnp.float32)
        outs.append(out_h)
    return jnp.concatenate(outs, axis=1) + bias


def _main_kernel(xb_ref, cntT_ref,
                 W0t_ref, b0_ref, g0_ref, be0_ref,
                 Wl1_ref, bl1_ref, Wr1_ref, br1_ref, af1_ref, sgn1_ref,
                 bias1_ref,
                 Wl2_ref, bl2_ref, Wr2_ref, br2_ref, af2_ref, sgn2_ref,
                 bias2_ref,
                 Wout_ref, bout_ref, g1_ref, be1_ref,
                 out_ref, xlpT_scr, xrpT_scr):
    eps = 1e-5
    inv = 1.0 / math.sqrt(1.0 + eps)
    xb = xb_ref[0]
    cntT = cntT_ref[...]
    neg_maskT = jnp.where(cntT > 0.0, 0.0, -jnp.inf)

    z = jnp.dot(xb, W0t_ref[...], preferred_element_type=jnp.float32)
    z = z + b0_ref[...]
    z = g0_ref[...] * (z * inv) + be0_ref[...]
    h = _gelu(z)

    h = _gat_layer(h, cntT, neg_maskT, Wl1_ref[...], bl1_ref[...],
                   Wr1_ref[...], br1_ref[...], sgn1_ref,
                   bias1_ref[...], xlpT_scr, xrpT_scr, af1_ref[...])
    h = _gelu(h)
    h = _gat_layer(h, cntT, neg_maskT, Wl2_ref[...], bl2_ref[...],
                   Wr2_ref[...], br2_ref[...], sgn2_ref,
                   bias2_ref[...], xlpT_scr, xrpT_scr, af2_ref[...])
    h = _gelu(h)

    pooled = jnp.mean(h, axis=0, keepdims=True)
    o = jnp.dot(pooled, Wout_ref[...], preferred_element_type=jnp.float32)
    o = o + bout_ref[...]
    o = g1_ref[...] * (o * inv) + be1_ref[...]
    out_ref[0] = _gelu(o)


def kernel(x, edge_index, W0, b0, g0, be0, Wl1, bl1, Wr1, br1, att1, bias1,
           Wl2, bl2, Wr2, br2, att2, bias2, Wout, bout, g1, be1):
    cntT = _edge_counts(edge_index)

    xb = x.reshape(B, N, N * T)
    row = lambda v: v.reshape(1, HID)
    full = lambda a: pl.BlockSpec(a.shape, lambda b: (0,) * a.ndim)
    signbit = jnp.int32(-2147483648)

    def sgn_col(att):
        af = att.reshape(HID)
        return jnp.where(af < 0.0, signbit, 0).astype(jnp.int32).reshape(HID, 1)

    args = [xb, cntT,
            W0.T, row(b0), row(g0), row(be0),
            Wl1, row(bl1), Wr1, row(br1), att1.reshape(1, HID),
            sgn_col(att1), row(bias1),
            Wl2, row(bl2), Wr2, row(br2), att2.reshape(1, HID),
            sgn_col(att2), row(bias2),
            Wout, row(bout), row(g1), row(be1)]
    in_specs = [pl.BlockSpec((1, N, N * T), lambda b: (b, 0, 0))]
    in_specs += [full(a) for a in args[1:]]

    out = pl.pallas_call(
        _main_kernel,
        grid=(B,),
        in_specs=in_specs,
        out_specs=pl.BlockSpec((1, 1, HID), lambda b: (b, 0, 0)),
        out_shape=jax.ShapeDtypeStruct((B, 1, HID), jnp.float32),
        scratch_shapes=[
            pltpu.VMEM((HID, N), jnp.float32),
            pltpu.VMEM((HID, N), jnp.float32),
        ],
        compiler_params=pltpu.CompilerParams(
            dimension_semantics=("parallel",)),
    )(*args)
    return out.reshape(B, HID)


# speedup vs baseline: 2.2511x; 1.1717x over previous
import math

import jax
import jax.numpy as jnp
from jax.experimental import pallas as pl
from jax.experimental.pallas import tpu as pltpu

B, N, T = 16, 512, 3
HID, H, C = 128, 4, 32
E = 65536

ECHUNK = 2048
NCHUNK = E // ECHUNK


def _gelu(v):
    return 0.5 * v * (1.0 + jax.lax.erf(v * (1.0 / math.sqrt(2.0))))


def _cnt_kernel(src_ref, dst_ref, out_ref):
    @pl.when(pl.program_id(0) == 0)
    def _():
        out_ref[...] = jnp.zeros_like(out_ref)

    s_blk = src_ref[0]
    d_blk = dst_ref[0]
    iota = jax.lax.broadcasted_iota(jnp.int32, (N, 1), 0)
    oh_s = (iota == s_blk).astype(jnp.bfloat16)
    oh_d = (iota == d_blk).astype(jnp.bfloat16)
    part = jax.lax.dot_general(
        oh_d, oh_s, (((1,), (1,)), ((), ())),
        preferred_element_type=jnp.float32)
    out_ref[...] += part


def _edge_counts(edge_index):
    src = edge_index[0].reshape(NCHUNK, 1, ECHUNK)
    dst = edge_index[1].reshape(NCHUNK, 1, ECHUNK)
    return pl.pallas_call(
        _cnt_kernel,
        grid=(NCHUNK,),
        in_specs=[
            pl.BlockSpec((1, 1, ECHUNK), lambda i: (i, 0, 0)),
            pl.BlockSpec((1, 1, ECHUNK), lambda i: (i, 0, 0)),
        ],
        out_specs=pl.BlockSpec((N, N), lambda i: (0, 0)),
        out_shape=jax.ShapeDtypeStruct((N, N), jnp.float32),
    )(src, dst)


def _gat_layer(h, cntT, neg_maskT, Wl, bl, Wr, br, sgn_ref, bias,
               xlpT_scr, xrpT_scr, att_flat):
    xl = jnp.dot(h, Wl, preferred_element_type=jnp.float32) + bl
    xr = jnp.dot(h, Wr, preferred_element_type=jnp.float32) + br
    xlpT_scr[...] = (xl * att_flat).T
    xrpT_scr[...] = (xr * att_flat).T
    ones_row = jnp.ones((1, N), jnp.float32)
    ones_c = jnp.ones((1, C), jnp.float32)
    absmask = jnp.int32(0x7FFFFFFF)
    outs = []
    for hd in range(H):
        xl_h = xl[:, hd * C:(hd + 1) * C]
        al_row = jnp.dot(ones_c, xlpT_scr[hd * C:(hd + 1) * C, :],
                         preferred_element_type=jnp.float32)
        ar_row = jnp.dot(ones_c, xrpT_scr[hd * C:(hd + 1) * C, :],
                         preferred_element_type=jnp.float32)
        lin = jax.lax.dot_general(
            jnp.concatenate([1.5 * ar_row, ones_row], axis=0),
            jnp.concatenate([ones_row, 1.5 * al_row], axis=0),
            (((0,), (0,)), ((), ())),
            preferred_element_type=jnp.float32)

        def body(c, acc):
            vp = xrpT_scr[pl.ds(hd * C + c, 1), :]
            up = xlpT_scr[pl.ds(hd * C + c, 1), :]
            tp = jax.lax.dot_general(
                jnp.concatenate([vp, ones_row], axis=0),
                jnp.concatenate([ones_row, up], axis=0),
                (((0,), (0,)), ((), ())),
                preferred_element_type=jnp.float32)
            ti = (jax.lax.bitcast_convert_type(tp, jnp.int32) & absmask) \
                ^ sgn_ref[pl.ds(hd * C + c, 1), 0:1]
            return acc + jax.lax.bitcast_convert_type(ti, jnp.float32)

        m = jax.lax.fori_loop(0, C, body, lin, unroll=8)
        alphaT = 0.4 * m
        amax = jnp.max(alphaT + neg_maskT, axis=1, keepdims=True)
        amax = jnp.where(jnp.isfinite(amax), amax, 0.0)
        ex = jnp.exp(jnp.minimum(alphaT - amax, 0.0))
        w = cntT * ex
        den = jnp.sum(w, axis=1, keepdims=True)
        wn = w * (1.0 / (den + 1e-16))
        out_h = jnp.dot(wn, xl_h, preferred_element_type=jnp.float32)
        outs.append(out_h)
    return jnp.concatenate(outs, axis=1) + bias


def _main_kernel(xb_ref, cntT_ref,
                 W0t_ref, b0_ref, g0_ref, be0_ref,
                 Wl1_ref, bl1_ref, Wr1_ref, br1_ref, af1_ref, sgn1_ref,
                 bias1_ref,
                 Wl2_ref, bl2_ref, Wr2_ref, br2_ref, af2_ref, sgn2_ref,
                 bias2_ref,
                 Wout_ref, bout_ref, g1_ref, be1_ref,
                 out_ref, xlpT_scr, xrpT_scr):
    eps = 1e-5
    inv = 1.0 / math.sqrt(1.0 + eps)
    xb = xb_ref[0]
    cntT = cntT_ref[...]
    neg_maskT = jnp.where(cntT > 0.0, 0.0, -jnp.inf)

    z = jnp.dot(xb, W0t_ref[...], preferred_element_type=jnp.float32)
    z = z + b0_ref[...]
    z = g0_ref[...] * (z * inv) + be0_ref[...]
    h = _gelu(z)

    h = _gat_layer(h, cntT, neg_maskT, Wl1_ref[...], bl1_ref[...],
                   Wr1_ref[...], br1_ref[...], sgn1_ref,
                   bias1_ref[...], xlpT_scr, xrpT_scr, af1_ref[...])
    h = _gelu(h)
    h = _gat_layer(h, cntT, neg_maskT, Wl2_ref[...], bl2_ref[...],
                   Wr2_ref[...], br2_ref[...], sgn2_ref,
                   bias2_ref[...], xlpT_scr, xrpT_scr, af2_ref[...])
    h = _gelu(h)

    pooled = jnp.mean(h, axis=0, keepdims=True)
    o = jnp.dot(pooled, Wout_ref[...], preferred_element_type=jnp.float32)
    o = o + bout_ref[...]
    o = g1_ref[...] * (o * inv) + be1_ref[...]
    out_ref[0] = _gelu(o)


def kernel(x, edge_index, W0, b0, g0, be0, Wl1, bl1, Wr1, br1, att1, bias1,
           Wl2, bl2, Wr2, br2, att2, bias2, Wout, bout, g1, be1):
    cntT = _edge_counts(edge_index)

    xb = x.reshape(B, N, N * T)
    row = lambda v: v.reshape(1, HID)
    full = lambda a: pl.BlockSpec(a.shape, lambda b: (0,) * a.ndim)
    signbit = jnp.int32(-2147483648)

    def sgn_col(att):
        af = att.reshape(HID)
        return jnp.where(af < 0.0, signbit, 0).astype(jnp.int32).reshape(HID, 1)

    args = [xb, cntT,
            W0.T, row(b0), row(g0), row(be0),
            Wl1, row(bl1), Wr1, row(br1), att1.reshape(1, HID),
            sgn_col(att1), row(bias1),
            Wl2, row(bl2), Wr2, row(br2), att2.reshape(1, HID),
            sgn_col(att2), row(bias2),
            Wout, row(bout), row(g1), row(be1)]
    in_specs = [pl.BlockSpec((1, N, N * T), lambda b: (b, 0, 0))]
    in_specs += [full(a) for a in args[1:]]

    out = pl.pallas_call(
        _main_kernel,
        grid=(B,),
        in_specs=in_specs,
        out_specs=pl.BlockSpec((1, 1, HID), lambda b: (b, 0, 0)),
        out_shape=jax.ShapeDtypeStruct((B, 1, HID), jnp.float32),
        scratch_shapes=[
            pltpu.VMEM((HID, N), jnp.float32),
            pltpu.VMEM((HID, N), jnp.float32),
        ],
        compiler_params=pltpu.CompilerParams(
            dimension_semantics=("parallel",)),
    )(*args)
    return out.reshape(B, HID)


# speedup vs baseline: 2.4609x; 1.0932x over previous
import math

import jax
import jax.numpy as jnp
from jax.experimental import pallas as pl
from jax.experimental.pallas import tpu as pltpu

B, N, T = 16, 512, 3
HID, H, C = 128, 4, 32
E = 65536

ECHUNK = 2048
NCHUNK = E // ECHUNK


def _gelu(v):
    return 0.5 * v * (1.0 + jax.lax.erf(v * (1.0 / math.sqrt(2.0))))


def _cnt_kernel(src_ref, dst_ref, out_ref):
    @pl.when(pl.program_id(0) == 0)
    def _():
        out_ref[...] = jnp.zeros_like(out_ref)

    s_blk = src_ref[0]
    d_blk = dst_ref[0]
    iota = jax.lax.broadcasted_iota(jnp.int32, (N, 1), 0)
    oh_s = (iota == s_blk).astype(jnp.bfloat16)
    oh_d = (iota == d_blk).astype(jnp.bfloat16)
    part = jax.lax.dot_general(
        oh_d, oh_s, (((1,), (1,)), ((), ())),
        preferred_element_type=jnp.float32)
    out_ref[...] += part


def _edge_counts(edge_index):
    src = edge_index[0].reshape(NCHUNK, 1, ECHUNK)
    dst = edge_index[1].reshape(NCHUNK, 1, ECHUNK)
    return pl.pallas_call(
        _cnt_kernel,
        grid=(NCHUNK,),
        in_specs=[
            pl.BlockSpec((1, 1, ECHUNK), lambda i: (i, 0, 0)),
            pl.BlockSpec((1, 1, ECHUNK), lambda i: (i, 0, 0)),
        ],
        out_specs=pl.BlockSpec((N, N), lambda i: (0, 0)),
        out_shape=jax.ShapeDtypeStruct((N, N), jnp.float32),
    )(src, dst)


def _gat_layer(h, cntT, neg_maskT, Wl, bl, Wr, br, sgn_ref, bias,
               xlpT_scr, xrpT_scr, att_flat):
    xl = jnp.dot(h, Wl, preferred_element_type=jnp.float32) + bl
    xr = jnp.dot(h, Wr, preferred_element_type=jnp.float32) + br
    xlpT_scr[...] = (xl * att_flat).T
    xrpT_scr[...] = (xr * att_flat).T
    ones_row = jnp.ones((1, N), jnp.float32)
    ones_c = jnp.ones((1, C), jnp.float32)
    absmask = jnp.int32(0x7FFFFFFF)
    outs = []
    for hd in range(H):
        xl_h = xl[:, hd * C:(hd + 1) * C]
        al_row = jnp.dot(ones_c, xlpT_scr[hd * C:(hd + 1) * C, :],
                         preferred_element_type=jnp.float32)
        ar_row = jnp.dot(ones_c, xrpT_scr[hd * C:(hd + 1) * C, :],
                         preferred_element_type=jnp.float32)
        lin = jax.lax.dot_general(
            jnp.concatenate([1.5 * ar_row, ones_row], axis=0),
            jnp.concatenate([ones_row, 1.5 * al_row], axis=0),
            (((0,), (0,)), ((), ())),
            preferred_element_type=jnp.float32)

        def body(c, acc):
            vp = xrpT_scr[pl.ds(hd * C + c, 1), :]
            up = xlpT_scr[pl.ds(hd * C + c, 1), :]
            tp = jax.lax.dot_general(
                jnp.concatenate([vp, ones_row], axis=0),
                jnp.concatenate([ones_row, up], axis=0),
                (((0,), (0,)), ((), ())),
                preferred_element_type=jnp.float32)
            ti = (jax.lax.bitcast_convert_type(tp, jnp.int32) & absmask) \
                ^ sgn_ref[pl.ds(hd * C + c, 1), 0:1]
            return acc + jax.lax.bitcast_convert_type(ti, jnp.float32)

        m = jax.lax.fori_loop(0, C, body, lin, unroll=16)
        alphaT = 0.4 * m
        amax = jnp.max(alphaT + neg_maskT, axis=1, keepdims=True)
        amax = jnp.where(jnp.isfinite(amax), amax, 0.0)
        ex = jnp.exp(jnp.minimum(alphaT - amax, 0.0))
        w = cntT * ex
        den = jnp.sum(w, axis=1, keepdims=True)
        wn = w * (1.0 / (den + 1e-16))
        out_h = jnp.dot(wn, xl_h, preferred_element_type=jnp.float32)
        outs.append(out_h)
    return jnp.concatenate(outs, axis=1) + bias


def _main_kernel(xb_ref, cntT_ref,
                 W0t_ref, b0_ref, g0_ref, be0_ref,
                 Wl1_ref, bl1_ref, Wr1_ref, br1_ref, af1_ref, sgn1_ref,
                 bias1_ref,
                 Wl2_ref, bl2_ref, Wr2_ref, br2_ref, af2_ref, sgn2_ref,
                 bias2_ref,
                 Wout_ref, bout_ref, g1_ref, be1_ref,
                 out_ref, xlpT_scr, xrpT_scr):
    eps = 1e-5
    inv = 1.0 / math.sqrt(1.0 + eps)
    xb = xb_ref[0]
    cntT = cntT_ref[...]
    neg_maskT = jnp.where(cntT > 0.0, 0.0, -jnp.inf)

    z = jnp.dot(xb, W0t_ref[...], preferred_element_type=jnp.float32)
    z = z + b0_ref[...]
    z = g0_ref[...] * (z * inv) + be0_ref[...]
    h = _gelu(z)

    h = _gat_layer(h, cntT, neg_maskT, Wl1_ref[...], bl1_ref[...],
                   Wr1_ref[...], br1_ref[...], sgn1_ref,
                   bias1_ref[...], xlpT_scr, xrpT_scr, af1_ref[...])
    h = _gelu(h)
    h = _gat_layer(h, cntT, neg_maskT, Wl2_ref[...], bl2_ref[...],
                   Wr2_ref[...], br2_ref[...], sgn2_ref,
                   bias2_ref[...], xlpT_scr, xrpT_scr, af2_ref[...])
    h = _gelu(h)

    pooled = jnp.mean(h, axis=0, keepdims=True)
    o = jnp.dot(pooled, Wout_ref[...], preferred_element_type=jnp.float32)
    o = o + bout_ref[...]
    o = g1_ref[...] * (o * inv) + be1_ref[...]
    out_ref[0] = _gelu(o)


def kernel(x, edge_index, W0, b0, g0, be0, Wl1, bl1, Wr1, br1, att1, bias1,
           Wl2, bl2, Wr2, br2, att2, bias2, Wout, bout, g1, be1):
    cntT = _edge_counts(edge_index)

    xb = x.reshape(B, N, N * T)
    row = lambda v: v.reshape(1, HID)
    full = lambda a: pl.BlockSpec(a.shape, lambda b: (0,) * a.ndim)
    signbit = jnp.int32(-2147483648)

    def sgn_col(att):
        af = att.reshape(HID)
        return jnp.where(af < 0.0, signbit, 0).astype(jnp.int32).reshape(HID, 1)

    args = [xb, cntT,
            W0.T, row(b0), row(g0), row(be0),
            Wl1, row(bl1), Wr1, row(br1), att1.reshape(1, HID),
            sgn_col(att1), row(bias1),
            Wl2, row(bl2), Wr2, row(br2), att2.reshape(1, HID),
            sgn_col(att2), row(bias2),
            Wout, row(bout), row(g1), row(be1)]
    in_specs = [pl.BlockSpec((1, N, N * T), lambda b: (b, 0, 0))]
    in_specs += [full(a) for a in args[1:]]

    out = pl.pallas_call(
        _main_kernel,
        grid=(B,),
        in_specs=in_specs,
        out_specs=pl.BlockSpec((1, 1, HID), lambda b: (b, 0, 0)),
        out_shape=jax.ShapeDtypeStruct((B, 1, HID), jnp.float32),
        scratch_shapes=[
            pltpu.VMEM((HID, N), jnp.float32),
            pltpu.VMEM((HID, N), jnp.float32),
        ],
        compiler_params=pltpu.CompilerParams(
            dimension_semantics=("parallel",)),
    )(*args)
    return out.reshape(B, HID)
